# baseline (speedup 1.0000x reference)
import functools

import jax
import jax.numpy as jnp
from jax import lax
from jax.experimental import pallas as pl
from jax.experimental.pallas import tpu as pltpu
from jax.experimental.pallas import tpu_sc as plsc

B, N, D, R = 8, 4096, 1024, 100
NSPLIT = 16
TN = N // NSPLIT

NW = 32
ROWS_PW = B * N // NW
CH = 16
CHUNKS = ROWS_PW // CH


def _sc_copy_kernel(src_hbm, out_hbm, buf0, buf1, sem0, sem1):
    wid = lax.axis_index("s") * 2 + lax.axis_index("c")
    base = wid * ROWS_PW
    bufs = (buf0, buf1)
    sems = (sem0, sem1)

    def do2(g2, _):
        g = g2 * 2
        c0 = pltpu.make_async_copy(
            src_hbm.at[pl.ds(base + g * CH, CH), :], buf0, sem0)
        c0.start()
        c1 = pltpu.make_async_copy(
            src_hbm.at[pl.ds(base + (g + 1) * CH, CH), :], buf1, sem1)
        c1.start()
        c0.wait()
        pltpu.sync_copy(buf0, out_hbm.at[pl.ds(base + g * CH, CH), :])
        c1.wait()
        pltpu.sync_copy(buf1, out_hbm.at[pl.ds(base + (g + 1) * CH, CH), :])
        return _

    lax.fori_loop(0, CHUNKS // 2, do2, 0)


@jax.jit
def _sc_copy(node2d):
    return pl.kernel(
        _sc_copy_kernel,
        mesh=plsc.VectorSubcoreMesh(core_axis_name="c", subcore_axis_name="s"),
        out_type=jax.ShapeDtypeStruct((B * N, D), jnp.float32),
        scratch_types=[
            pltpu.VMEM((CH, D), jnp.float32),
            pltpu.VMEM((CH, D), jnp.float32),
            pltpu.SemaphoreType.DMA,
            pltpu.SemaphoreType.DMA,
        ],
    )(node2d)


def _flash_kernel(*refs):
    n_refs = refs[:NSPLIT]
    edge_ref, q_ref, out_ref = refs[NSPLIT:]
    q = q_ref[...].astype(jnp.bfloat16)
    nbs = []
    logits = []
    for h, nr in enumerate(n_refs):
        nb = nr[0].astype(jnp.bfloat16)
        nbs.append(nb)
        l = jax.lax.dot_general(
            nb, q, (((1,), (1,)), ((), ())),
            preferred_element_type=jnp.float32)
        mask = edge_ref[0, h * TN:(h + 1) * TN] == 1
        logits.append(jnp.where(mask, l, jnp.float32(-1e30)))

    m = jnp.max(logits[0], axis=0, keepdims=True)
    for l in logits[1:]:
        m = jnp.maximum(m, jnp.max(l, axis=0, keepdims=True))

    s = jnp.zeros((1, R), jnp.float32)
    acc = jnp.zeros((R, D), jnp.float32)
    for l, nb in zip(logits, nbs):
        e = jnp.exp(l - m)
        s = s + jnp.sum(e, axis=0, keepdims=True)
        acc = acc + jax.lax.dot_general(
            e.astype(jnp.bfloat16), nb, (((0,), (0,)), ((), ())),
            preferred_element_type=jnp.float32)
    out_ref[0] = acc / s.T


@jax.jit
def _run(node_feature, edge_weight, relation_weight):
    edge3 = edge_weight.reshape(B, N, 1)
    nspec = [
        pl.BlockSpec((1, TN, D), lambda b, h=h: (b, h, 0)) for h in range(NSPLIT)
    ]
    return pl.pallas_call(
        _flash_kernel,
        grid=(B,),
        in_specs=nspec + [
            pl.BlockSpec((1, N, 1), lambda b: (b, 0, 0)),
            pl.BlockSpec((R, D), lambda b: (0, 0)),
        ],
        out_specs=pl.BlockSpec((1, R, D), lambda b: (b, 0, 0)),
        out_shape=jax.ShapeDtypeStruct((B, R, D), jnp.float32),
        compiler_params=pltpu.CompilerParams(
            dimension_semantics=("arbitrary",),
        ),
    )(*([node_feature] * NSPLIT), edge3, relation_weight)


def kernel(node_feature, edge_weight, index, mention_count, relation_label,
           is_train, relation_weight):
    out = _run(node_feature, edge_weight, relation_weight)
    sc_out = _sc_copy(node_feature.reshape(B * N, D))
    return out + jnp.float32(0.0) * sc_out[0, 0]


# speedup vs baseline: 2.5601x; 2.5601x over previous
import jax
import jax.numpy as jnp
from jax.experimental import pallas as pl
from jax.experimental.pallas import tpu as pltpu

B, N, D, R = 8, 4096, 1024, 100
NSPLIT = 16
TN = N // NSPLIT


def _flash_kernel(*refs):
    n_refs = refs[:NSPLIT]
    edge_ref, q_ref, out_ref = refs[NSPLIT:]
    q = q_ref[...].astype(jnp.bfloat16)
    nbs = []
    logits = []
    for h, nr in enumerate(n_refs):
        nb = nr[0].astype(jnp.bfloat16)
        nbs.append(nb)
        l = jax.lax.dot_general(
            nb, q, (((1,), (1,)), ((), ())),
            preferred_element_type=jnp.float32)
        mask = edge_ref[0, h * TN:(h + 1) * TN] == 1
        logits.append(jnp.where(mask, l, jnp.float32(-1e30)))

    m = jnp.max(logits[0], axis=0, keepdims=True)
    for l in logits[1:]:
        m = jnp.maximum(m, jnp.max(l, axis=0, keepdims=True))

    s = jnp.zeros((1, R), jnp.float32)
    acc = jnp.zeros((R, D), jnp.float32)
    for l, nb in zip(logits, nbs):
        e = jnp.exp(l - m)
        s = s + jnp.sum(e, axis=0, keepdims=True)
        acc = acc + jax.lax.dot_general(
            e.astype(jnp.bfloat16), nb, (((0,), (0,)), ((), ())),
            preferred_element_type=jnp.float32)
    out_ref[0] = acc / s.T


@jax.jit
def _run(node_feature, edge_weight, relation_weight):
    edge3 = edge_weight.reshape(B, N, 1)
    nspec = [
        pl.BlockSpec((1, TN, D), lambda b, h=h: (b, h, 0)) for h in range(NSPLIT)
    ]
    return pl.pallas_call(
        _flash_kernel,
        grid=(B,),
        in_specs=nspec + [
            pl.BlockSpec((1, N, 1), lambda b: (b, 0, 0)),
            pl.BlockSpec((R, D), lambda b: (0, 0)),
        ],
        out_specs=pl.BlockSpec((1, R, D), lambda b: (b, 0, 0)),
        out_shape=jax.ShapeDtypeStruct((B, R, D), jnp.float32),
        compiler_params=pltpu.CompilerParams(
            dimension_semantics=("arbitrary",),
        ),
    )(*([node_feature] * NSPLIT), edge3, relation_weight)


def kernel(node_feature, edge_weight, index, mention_count, relation_label,
           is_train, relation_weight):
    return _run(node_feature, edge_weight, relation_weight)
